# baseline (speedup 1.0000x reference)
import functools

import jax
import jax.numpy as jnp
from jax import lax
from jax.experimental import pallas as pl
from jax.experimental.pallas import tpu as pltpu
from jax.experimental.pallas import tpu_sc as plsc

D = 64
HIDDEN = 128
OUT_DIM = 32
V = 100000
VQP = 25088
BV = 3584


def _precompute_body(e0, e1, e2, e3, w1, b1, w2, b2, o_ref):
    for k, e in enumerate((e0, e1, e2, e3)):
        et = jnp.transpose(e[...])
        h = jnp.dot(et, w1[...], preferred_element_type=jnp.float32)
        h = jnp.maximum(h + b1[...], 0.0)
        o = jnp.dot(h, w2[...], preferred_element_type=jnp.float32)
        o_ref[:, 32 * k:32 * (k + 1)] = o + b2[...]


def _precompute(emb_t, W1t, b1t, W2t, b2t):
    grid = (VQP // BV,)
    nb = VQP // BV
    e_spec = lambda k: pl.BlockSpec((D, BV), lambda i, k=k: (0, nb * k + i))
    return pl.pallas_call(
        _precompute_body,
        grid=grid,
        in_specs=[
            e_spec(0), e_spec(1), e_spec(2), e_spec(3),
            pl.BlockSpec((D, HIDDEN), lambda i: (0, 0)),
            pl.BlockSpec((1, HIDDEN), lambda i: (0, 0)),
            pl.BlockSpec((HIDDEN, OUT_DIM), lambda i: (0, 0)),
            pl.BlockSpec((1, OUT_DIM), lambda i: (0, 0)),
        ],
        out_specs=pl.BlockSpec((BV, HIDDEN), lambda i: (i, 0)),
        out_shape=jax.ShapeDtypeStruct((VQP, HIDDEN), jnp.float32),
    )(emb_t, emb_t, emb_t, emb_t, W1t, b1t, W2t, b2t)


@functools.lru_cache(maxsize=None)
def _make_gather(B):
    info = plsc.get_sparse_core_info()
    NC, NS = info.num_cores, info.num_subcores
    NW = NC * NS
    b_per_w = B // NW
    mesh = plsc.VectorSubcoreMesh(core_axis_name="c", subcore_axis_name="s")

    n_chunks = 4
    c_rows = b_per_w // n_chunks

    @functools.partial(
        pl.kernel,
        mesh=mesh,
        out_type=jax.ShapeDtypeStruct((OUT_DIM, B), jnp.float32),
        scratch_types=[
            pltpu.VMEM((b_per_w,), jnp.int32),
            pltpu.VMEM((b_per_w,), jnp.int32),
            pltpu.VMEM((b_per_w, HIDDEN), jnp.float32),
            pltpu.VMEM((OUT_DIM, b_per_w), jnp.float32),
            pltpu.SemaphoreType.DMA,
            pltpu.SemaphoreType.DMA,
            pltpu.SemaphoreType.DMA,
            pltpu.SemaphoreType.DMA,
        ],
        compiler_params=pltpu.CompilerParams(needs_layout_passes=False),
    )
    def gather_k(table_hbm, idx_hbm, outT_hbm, idx_v, idx2_v, rows_v, xt_v,
                 *sems):
        wid = lax.axis_index("s") * NC + lax.axis_index("c")
        base = wid * b_per_w
        pltpu.sync_copy(idx_hbm.at[pl.ds(base, b_per_w)], idx_v)

        def mod_body(j, carry):
            sl = pl.ds(j * 16, 16)
            idx2_v[sl] = lax.rem(idx_v[sl], VQP)
            return carry

        lax.fori_loop(0, b_per_w // 16, mod_body, 0)
        for c in range(n_chunks):
            pltpu.async_copy(
                table_hbm.at[idx2_v.at[pl.ds(c * c_rows, c_rows)]],
                rows_v.at[pl.ds(c * c_rows, c_rows)],
                sems[c],
            )

        def sel_body(g, carry):
            sl = pl.ds(g * 16, 16)
            iv = idx_v[sl]
            q = lax.shift_right_logical(
                lax.shift_right_logical(iv, 9) * 1338, 16)
            q32 = q * 32
            rvec = lax.iota(jnp.int32, 16) + g * 16
            for d in range(OUT_DIM):
                xt_v[d, sl] = plsc.load_gather(rows_v, [rvec, q32 + d])
            return carry

        for c in range(n_chunks):
            pltpu.make_async_copy(
                table_hbm.at[idx2_v.at[pl.ds(c * c_rows, c_rows)]],
                rows_v.at[pl.ds(c * c_rows, c_rows)],
                sems[c],
            ).wait()
            lax.fori_loop(c * (c_rows // 16), (c + 1) * (c_rows // 16),
                          sel_body, 0)
        pltpu.sync_copy(xt_v, outT_hbm.at[:, pl.ds(base, b_per_w)])

    return gather_k


@jax.jit
def kernel(gene_ids, emb, W1, b1, W2, b2):
    idx = gene_ids.astype(jnp.int32)
    B = idx.shape[0]
    emb_t = jnp.transpose(emb)
    table = _precompute(emb_t, W1, b1.reshape(1, HIDDEN),
                        W2, b2.reshape(1, OUT_DIM))
    outT = _make_gather(B)(table, idx)
    return jnp.transpose(outT)
